# baseline (speedup 1.0000x reference)
import functools

import jax
import jax.numpy as jnp
from jax import lax
from jax.experimental import pallas as pl
from jax.experimental.pallas import tpu as pltpu
from jax.experimental.pallas import tpu_sc as plsc

LUT_DIM = 64
_NC = 2
_NS = 16
_NW = _NC * _NS
_CHUNK = 800
_RING = 2
_IRING = 4


@functools.lru_cache(maxsize=None)
def _make_gather(b_total, dim):
    assert b_total % _NW == 0
    n_per_w = b_total // _NW
    chunk = _CHUNK
    ring = _RING
    assert n_per_w % chunk == 0
    n_chunks = n_per_w // chunk
    iring = _IRING
    assert n_chunks >= max(ring, iring)

    mesh = plsc.VectorSubcoreMesh(core_axis_name="c", subcore_axis_name="s")

    @functools.partial(
        pl.kernel,
        mesh=mesh,
        out_type=jax.ShapeDtypeStruct((b_total, dim), jnp.float32),
        scratch_types=[
            pltpu.VMEM((iring, chunk), jnp.int32),
            pltpu.VMEM((ring, chunk, dim), jnp.float32),
            pltpu.SemaphoreType.DMA,
            pltpu.SemaphoreType.DMA,
            pltpu.SemaphoreType.DMA,
        ],
        compiler_params=pltpu.CompilerParams(use_tc_tiling_on_sc=False),
    )
    def gather_kernel(idx_hbm, table_hbm, out_hbm, idx_v, rows_v, isem, gsem, wsem):
        cid = lax.axis_index("c")
        sid = lax.axis_index("s")
        wid = sid * _NC + cid
        base = wid * n_per_w

        def start_idx(c):
            pltpu.async_copy(
                idx_hbm.at[wid, c], idx_v.at[lax.rem(c, iring)], isem
            )

        def wait_idx(c):
            pltpu.make_async_copy(
                idx_hbm.at[wid, c], idx_v.at[lax.rem(c, iring)], isem
            ).wait()

        def start_gather(c):
            pltpu.async_copy(
                table_hbm.at[idx_v.at[lax.rem(c, iring)]],
                rows_v.at[lax.rem(c, ring)],
                gsem,
            )

        def wait_gather(c):
            pltpu.make_async_copy(
                table_hbm.at[idx_v.at[lax.rem(c, iring)]],
                rows_v.at[lax.rem(c, ring)],
                gsem,
            ).wait()

        def start_write(c):
            pltpu.async_copy(
                rows_v.at[lax.rem(c, ring)],
                out_hbm.at[pl.ds(base + c * chunk, chunk)],
                wsem,
            )

        def wait_write(c):
            pltpu.make_async_copy(
                rows_v.at[lax.rem(c, ring)],
                out_hbm.at[pl.ds(base + c * chunk, chunk)],
                wsem,
            ).wait()

        start_idx(0)
        start_idx(1)
        wait_idx(0)
        start_gather(0)

        def body(i, carry):
            @pl.when(i + 2 < n_chunks)
            def _():
                start_idx(i + 2)

            @pl.when(i + 1 < n_chunks)
            def _():
                @pl.when(i + 1 >= ring)
                def _():
                    wait_write(i + 1 - ring)

                wait_idx(i + 1)
                start_gather(i + 1)

            wait_gather(i)
            start_write(i)
            return carry

        lax.fori_loop(0, n_chunks, body, 0)
        for c in range(n_chunks - ring, n_chunks):
            wait_write(c)

    return gather_kernel


def kernel(index, table):
    b, f = index.shape
    dim = table.shape[1]
    flat_idx = index.reshape(b * f).astype(jnp.int32)
    n_per_w = (b * f) // _NW
    idx3 = flat_idx.reshape(_NW, n_per_w // _CHUNK, _CHUNK)
    out = _make_gather(b * f, dim)(idx3, table)
    return out.reshape(b, f, dim)
